# baseline (speedup 1.0000x reference)
import functools

import jax
import jax.numpy as jnp
from jax import lax
from jax.experimental import pallas as pl
from jax.experimental.pallas import tpu as pltpu
from jax.experimental.pallas import tpu_sc as plsc

N_NODES = 100000
N_EDGES = 1600000
OUT_C = 16
LANES = 128
CHUNK_ROWS = 4
CHUNK_E = CHUNK_ROWS * LANES
E_ROWS = N_EDGES // LANES
FULL_CHUNKS = E_ROWS // CHUNK_ROWS
A_PAIRS = 49
ACC_ROWS = 100096
ZROWS = ACC_ROWS // 16
P_ROWS = N_NODES + 8
OROWS = 6256
OROWS_LAST = P_ROWS - 15 * OROWS
B_CHUNK = 128
B_FULL = N_NODES // B_CHUNK
B_TAIL = N_NODES - B_FULL * B_CHUNK
B_PAIRS = 13


def _sc_accumulate(edge3, wt):
    mesh = plsc.VectorSubcoreMesh(core_axis_name="c", subcore_axis_name="s")

    @functools.partial(
        pl.kernel,
        mesh=mesh,
        out_type=(
            jax.ShapeDtypeStruct((P_ROWS, OUT_C), jnp.float32),
            jax.ShapeDtypeStruct((P_ROWS, OUT_C), jnp.float32),
            jax.ShapeDtypeStruct((32, OUT_C), jnp.int32),
        ),
        scratch_types=[
            pltpu.VMEM((CHUNK_ROWS, LANES), jnp.int32),
            pltpu.VMEM((CHUNK_ROWS, LANES), jnp.int32),
            pltpu.VMEM((CHUNK_ROWS, LANES), jnp.int32),
            pltpu.VMEM((CHUNK_ROWS, LANES), jnp.int32),
            pltpu.VMEM((CHUNK_E, OUT_C), jnp.float32),
            pltpu.VMEM((CHUNK_E, OUT_C), jnp.float32),
            pltpu.VMEM((OUT_C,), jnp.int32),
            pltpu.VMEM_SHARED((ACC_ROWS, OUT_C), jnp.float32),
            pltpu.SemaphoreType.DMA,
            pltpu.SemaphoreType.DMA,
            pltpu.SemaphoreType.DMA,
            pltpu.SemaphoreType.DMA,
            pltpu.SemaphoreType.DMA,
            pltpu.SemaphoreType.DMA,
        ],
        compiler_params=pltpu.CompilerParams(use_tc_tiling_on_sc=False),
    )
    def k(edge_hbm, wt_hbm, p0_hbm, p1_hbm, mins_hbm,
          row_a, col_a, row_b, col_b, gbuf_a, gbuf_b, minbuf, acc,
          semi_a, semi_b, semg_a, semg_b, sems_a, sems_b):
        c = lax.axis_index("c")
        s = lax.axis_index("s")
        w = c * 16 + s

        bufs_a = (row_a, col_a, gbuf_a, semi_a, semg_a, sems_a)
        bufs_b = (row_b, col_b, gbuf_b, semi_b, semg_b, sems_b)

        def fire_idx(i, bufs):
            row_r, col_r, _, semi, _, _ = bufs
            base = pl.multiple_of((i * 32 + w) * CHUNK_ROWS, 8)
            pltpu.async_copy(edge_hbm.at[0, pl.ds(base, CHUNK_ROWS)], row_r, semi)
            pltpu.async_copy(edge_hbm.at[1, pl.ds(base, CHUNK_ROWS)], col_r, semi)

        def wait_idx(bufs):
            row_r, col_r, _, semi, _, _ = bufs
            pltpu.make_async_copy(edge_hbm.at[0, pl.ds(0, CHUNK_ROWS)],
                                  row_r, semi).wait()
            pltpu.make_async_copy(edge_hbm.at[1, pl.ds(0, CHUNK_ROWS)],
                                  col_r, semi).wait()

        def fire_gat(bufs):
            _, col_r, gb, _, semg, _ = bufs
            for j in range(CHUNK_ROWS):
                pltpu.async_copy(wt_hbm.at[col_r.at[j]],
                                 gb.at[pl.ds(j * LANES, LANES)], semg)

        def wait_gat(bufs):
            _, col_r, gb, _, semg, _ = bufs
            for j in range(CHUNK_ROWS):
                pltpu.make_async_copy(wt_hbm.at[col_r.at[j]],
                                      gb.at[pl.ds(j * LANES, LANES)],
                                      semg).wait()

        def fire_scat(bufs):
            row_r, _, gb, _, _, sems = bufs
            for j in range(CHUNK_ROWS):
                pltpu.async_copy(gb.at[pl.ds(j * LANES, LANES)],
                                 acc.at[row_r.at[j]], sems, add=True)

        def wait_scat(bufs):
            row_r, _, gb, _, _, sems = bufs
            for j in range(CHUNK_ROWS):
                pltpu.make_async_copy(gb.at[pl.ds(j * LANES, LANES)],
                                      acc.at[row_r.at[j]], sems).wait()

        def min_update(bufs):
            row_r = bufs[0]
            mv = minbuf[...]
            for j in range(CHUNK_ROWS):
                for kk in range(LANES // OUT_C):
                    mv = jnp.minimum(mv, row_r[j, pl.ds(kk * OUT_C, OUT_C)])
            minbuf[...] = mv

        def valid(i):
            return jnp.logical_and(i >= 0, i * 32 + w < FULL_CHUNKS)

        def zfill(i, carry):
            gbuf_a[i, :] = jnp.zeros((OUT_C,), jnp.float32)
            return carry

        lax.fori_loop(0, CHUNK_E, zfill, 0)
        zbase = s * ZROWS
        nfull = ZROWS // CHUNK_E
        for kk in range(nfull):
            pltpu.sync_copy(gbuf_a, acc.at[pl.ds(zbase + kk * CHUNK_E, CHUNK_E)])
        rem = ZROWS % CHUNK_E
        if rem:
            pltpu.sync_copy(gbuf_a.at[pl.ds(0, rem)],
                            acc.at[pl.ds(zbase + nfull * CHUNK_E, rem)])
        plsc.subcore_barrier()

        minbuf[...] = jnp.full((OUT_C,), jnp.int32(N_NODES), jnp.int32)

        fire_idx(0, bufs_a)
        wait_idx(bufs_a)
        fire_gat(bufs_a)

        def pair(g, carry):
            i0 = 2 * g
            i1 = 2 * g + 1

            @pl.when(valid(i1 - 2))
            def _():
                wait_scat(bufs_b)

            @pl.when(valid(i1))
            def _():
                fire_idx(i1, bufs_b)

            @pl.when(valid(i0))
            def _():
                wait_gat(bufs_a)
                min_update(bufs_a)
                fire_scat(bufs_a)

            @pl.when(valid(i1))
            def _():
                wait_idx(bufs_b)
                fire_gat(bufs_b)

            @pl.when(valid(i0))
            def _():
                wait_scat(bufs_a)

            @pl.when(valid(i0 + 2))
            def _():
                fire_idx(i0 + 2, bufs_a)

            @pl.when(valid(i1))
            def _():
                wait_gat(bufs_b)
                min_update(bufs_b)
                fire_scat(bufs_b)

            @pl.when(valid(i0 + 2))
            def _():
                wait_idx(bufs_a)
                fire_gat(bufs_a)

            return carry

        lax.fori_loop(0, A_PAIRS, pair, 0)

        @pl.when(valid(2 * A_PAIRS - 1))
        def _():
            wait_scat(bufs_b)

        pltpu.sync_copy(minbuf, mins_hbm.at[w])
        plsc.subcore_barrier()

        obase = pl.multiple_of(s * OROWS, 8)

        def copy_out(dst):
            @pl.when(s < 15)
            def _full():
                pltpu.sync_copy(acc.at[pl.ds(obase, OROWS)],
                                dst.at[pl.ds(obase, OROWS)])

            @pl.when(s == 15)
            def _last():
                pltpu.sync_copy(acc.at[pl.ds(15 * OROWS, OROWS_LAST)],
                                dst.at[pl.ds(15 * OROWS, OROWS_LAST)])

        @pl.when(c == 0)
        def _p0():
            copy_out(p0_hbm)

        @pl.when(c == 1)
        def _p1():
            copy_out(p1_hbm)

    return k(edge3, wt)


def _sc_merge(p0, p1, mins, bias):
    mesh = plsc.VectorSubcoreMesh(core_axis_name="c", subcore_axis_name="s")

    @functools.partial(
        pl.kernel,
        mesh=mesh,
        out_type=jax.ShapeDtypeStruct((N_NODES // 8, 8 * OUT_C), jnp.float32),
        scratch_types=[
            pltpu.VMEM((B_CHUNK,), jnp.int32),
            pltpu.VMEM((B_CHUNK,), jnp.int32),
            pltpu.VMEM((B_CHUNK, OUT_C), jnp.float32),
            pltpu.VMEM((B_CHUNK, OUT_C), jnp.float32),
            pltpu.VMEM((B_CHUNK, OUT_C), jnp.float32),
            pltpu.VMEM((B_CHUNK, OUT_C), jnp.float32),
            pltpu.VMEM((B_CHUNK // 8, 8 * OUT_C), jnp.float32),
            pltpu.VMEM((B_CHUNK // 8, 8 * OUT_C), jnp.float32),
            pltpu.VMEM((32, OUT_C), jnp.int32),
            pltpu.VMEM((OUT_C,), jnp.float32),
            pltpu.SemaphoreType.DMA,
            pltpu.SemaphoreType.DMA,
            pltpu.SemaphoreType.DMA,
            pltpu.SemaphoreType.DMA,
        ],
        compiler_params=pltpu.CompilerParams(use_tc_tiling_on_sc=False,
                                             needs_layout_passes=False),
    )
    def k(p0_hbm, p1_hbm, mins_hbm, bias_hbm, out_hbm,
          idx_a, idx_b, b0_a, b1_a, b0_b, b1_b, o_a, o_b, mbuf, bbuf,
          semg_a, semg_b, semo_a, semo_b):
        c = lax.axis_index("c")
        s = lax.axis_index("s")
        w = c * 16 + s

        pltpu.sync_copy(mins_hbm, mbuf)
        pltpu.sync_copy(bias_hbm, bbuf)
        mv = mbuf[0, :]
        for j in range(1, 32):
            mv = jnp.minimum(mv, mbuf[j, :])
        m = jnp.min(mv)
        mvec = jnp.full((OUT_C,), m, jnp.int32)
        bias_v = bbuf[...]
        lane = lax.iota(jnp.int32, OUT_C)

        bufs_a = (idx_a, b0_a, b1_a, o_a, semg_a, semo_a)
        bufs_b = (idx_b, b0_b, b1_b, o_b, semg_b, semo_b)

        def build_idx(i, bufs):
            idx_r = bufs[0]
            base = (i * 32 + w) * B_CHUNK
            for kk in range(B_CHUNK // OUT_C):
                iv = lane + (base + kk * OUT_C) + mvec
                idx_r[pl.ds(kk * OUT_C, OUT_C)] = jnp.minimum(
                    iv, jnp.int32(N_NODES))

        def fire_gat(bufs):
            idx_r, b0, b1, _, semg, _ = bufs
            pltpu.async_copy(p0_hbm.at[idx_r], b0, semg)
            pltpu.async_copy(p1_hbm.at[idx_r], b1, semg)

        def wait_gat(bufs):
            idx_r, b0, b1, _, semg, _ = bufs
            pltpu.make_async_copy(p0_hbm.at[idx_r], b0, semg).wait()
            pltpu.make_async_copy(p1_hbm.at[idx_r], b1, semg).wait()

        def add_rows(bufs):
            _, b0, b1, ob, _, _ = bufs

            def blk(t, carry):
                for r in range(8):
                    kk = t * 8 + r
                    ob[t, pl.ds(r * OUT_C, OUT_C)] = (
                        b0[kk, :] + b1[kk, :] + bias_v)
                return carry

            lax.fori_loop(0, B_CHUNK // 8, blk, 0)

        def fire_out(i, bufs):
            _, _, _, ob, _, semo = bufs
            base = pl.multiple_of((i * 32 + w) * (B_CHUNK // 8), 8)
            pltpu.async_copy(ob, out_hbm.at[pl.ds(base, B_CHUNK // 8)], semo)

        def wait_out(bufs):
            _, _, _, ob, _, semo = bufs
            pltpu.make_async_copy(ob, out_hbm.at[pl.ds(0, B_CHUNK // 8)],
                                  semo).wait()

        def valid(i):
            return jnp.logical_and(i >= 0, i * 32 + w < B_FULL)

        build_idx(0, bufs_a)
        fire_gat(bufs_a)

        def pair(g, carry):
            i0 = 2 * g
            i1 = 2 * g + 1

            @pl.when(valid(i1))
            def _():
                build_idx(i1, bufs_b)

            @pl.when(valid(i1 - 2))
            def _():
                wait_out(bufs_b)

            @pl.when(valid(i1))
            def _():
                fire_gat(bufs_b)

            @pl.when(valid(i0))
            def _():
                wait_gat(bufs_a)
                add_rows(bufs_a)
                fire_out(i0, bufs_a)

            @pl.when(valid(i0 + 2))
            def _():
                build_idx(i0 + 2, bufs_a)

            @pl.when(valid(i0))
            def _():
                wait_out(bufs_a)

            @pl.when(valid(i0 + 2))
            def _():
                fire_gat(bufs_a)

            @pl.when(valid(i1))
            def _():
                wait_gat(bufs_b)
                add_rows(bufs_b)
                fire_out(i1, bufs_b)

            return carry

        lax.fori_loop(0, B_PAIRS, pair, 0)

        @pl.when(valid(2 * B_PAIRS - 1))
        def _():
            wait_out(bufs_b)

        @pl.when(w == 13)
        def _tail():
            base = B_FULL * B_CHUNK
            for kk in range(B_TAIL // OUT_C):
                iv = lane + (base + kk * OUT_C) + mvec
                idx_a[pl.ds(kk * OUT_C, OUT_C)] = jnp.minimum(
                    iv, jnp.int32(N_NODES))
            cp0 = pltpu.async_copy(p0_hbm.at[idx_a.at[pl.ds(0, B_TAIL)]],
                                   b0_a.at[pl.ds(0, B_TAIL)], semg_a)
            cp1 = pltpu.async_copy(p1_hbm.at[idx_a.at[pl.ds(0, B_TAIL)]],
                                   b1_a.at[pl.ds(0, B_TAIL)], semg_a)
            cp0.wait()
            cp1.wait()
            for kk in range(B_TAIL):
                o_a[kk // 8, pl.ds((kk % 8) * OUT_C, OUT_C)] = (
                    b0_a[kk, :] + b1_a[kk, :] + bias_v)
            pltpu.sync_copy(o_a.at[pl.ds(0, B_TAIL // 8)],
                            out_hbm.at[pl.ds(base // 8, B_TAIL // 8)])

    return k(p0, p1, mins, bias)


def kernel(edge_index, W_weight, W_bias):
    edge3 = edge_index.reshape(2, E_ROWS, LANES)
    wt = W_weight.T
    p0, p1, mins = _sc_accumulate(edge3, wt)
    out2d = _sc_merge(p0, p1, mins, W_bias)
    return out2d.reshape(N_NODES, OUT_C)


# speedup vs baseline: 1.0533x; 1.0533x over previous
import functools

import jax
import jax.numpy as jnp
from jax import lax
from jax.experimental import pallas as pl
from jax.experimental.pallas import tpu as pltpu
from jax.experimental.pallas import tpu_sc as plsc

N_NODES = 100000
N_EDGES = 1600000
OUT_C = 16
LANES = 128
CHUNK_ROWS = 4
CHUNK_E = CHUNK_ROWS * LANES
E_ROWS = N_EDGES // LANES
FULL_CHUNKS = E_ROWS // CHUNK_ROWS
A_PAIRS = 49
ACC_ROWS = 100096
ZROWS = ACC_ROWS // 16
P_ROWS = N_NODES + 8
OROWS = 6256
OROWS_LAST = P_ROWS - 15 * OROWS
B_CHUNK = 128
B_FULL = N_NODES // B_CHUNK
B_TAIL = N_NODES - B_FULL * B_CHUNK
B_PAIRS = 13


def _sc_accumulate(edge3, wt):
    mesh = plsc.VectorSubcoreMesh(core_axis_name="c", subcore_axis_name="s")

    @functools.partial(
        pl.kernel,
        mesh=mesh,
        out_type=(
            jax.ShapeDtypeStruct((P_ROWS, OUT_C), jnp.float32),
            jax.ShapeDtypeStruct((P_ROWS, OUT_C), jnp.float32),
            jax.ShapeDtypeStruct((32, OUT_C), jnp.int32),
        ),
        scratch_types=[
            pltpu.VMEM((CHUNK_ROWS, LANES), jnp.int32),
            pltpu.VMEM((CHUNK_ROWS, LANES), jnp.int32),
            pltpu.VMEM((CHUNK_ROWS, LANES), jnp.int32),
            pltpu.VMEM((CHUNK_ROWS, LANES), jnp.int32),
            pltpu.VMEM((CHUNK_E, OUT_C), jnp.float32),
            pltpu.VMEM((CHUNK_E, OUT_C), jnp.float32),
            pltpu.VMEM((OUT_C,), jnp.int32),
            pltpu.VMEM_SHARED((ACC_ROWS, OUT_C), jnp.float32),
            pltpu.SemaphoreType.DMA,
            pltpu.SemaphoreType.DMA,
            pltpu.SemaphoreType.DMA,
            pltpu.SemaphoreType.DMA,
            pltpu.SemaphoreType.DMA,
            pltpu.SemaphoreType.DMA,
        ],
        compiler_params=pltpu.CompilerParams(use_tc_tiling_on_sc=False),
    )
    def k(edge_hbm, wt_hbm, p0_hbm, p1_hbm, mins_hbm,
          row_a, col_a, row_b, col_b, gbuf_a, gbuf_b, minbuf, acc,
          semi_a, semi_b, semg_a, semg_b, sems_a, sems_b):
        c = lax.axis_index("c")
        s = lax.axis_index("s")
        w = c * 16 + s

        bufs_a = (row_a, col_a, gbuf_a, semi_a, semg_a, sems_a)
        bufs_b = (row_b, col_b, gbuf_b, semi_b, semg_b, sems_b)

        def fire_idx(i, bufs):
            row_r, col_r, _, semi, _, _ = bufs
            base = pl.multiple_of((i * 32 + w) * CHUNK_ROWS, 8)
            pltpu.async_copy(edge_hbm.at[0, pl.ds(base, CHUNK_ROWS)], row_r, semi)
            pltpu.async_copy(edge_hbm.at[1, pl.ds(base, CHUNK_ROWS)], col_r, semi)

        def wait_idx(bufs):
            row_r, col_r, _, semi, _, _ = bufs
            pltpu.make_async_copy(edge_hbm.at[0, pl.ds(0, CHUNK_ROWS)],
                                  row_r, semi).wait()
            pltpu.make_async_copy(edge_hbm.at[1, pl.ds(0, CHUNK_ROWS)],
                                  col_r, semi).wait()

        def fire_gat(bufs):
            _, col_r, gb, _, semg, _ = bufs
            for j in range(CHUNK_ROWS):
                pltpu.async_copy(wt_hbm.at[col_r.at[j]],
                                 gb.at[pl.ds(j * LANES, LANES)], semg)

        def wait_gat(bufs):
            _, col_r, gb, _, semg, _ = bufs
            for j in range(CHUNK_ROWS):
                pltpu.make_async_copy(wt_hbm.at[col_r.at[j]],
                                      gb.at[pl.ds(j * LANES, LANES)],
                                      semg).wait()

        def fire_scat(bufs):
            row_r, _, gb, _, _, sems = bufs
            for j in range(CHUNK_ROWS):
                pltpu.async_copy(gb.at[pl.ds(j * LANES, LANES)],
                                 acc.at[row_r.at[j]], sems, add=True)

        def wait_scat(bufs):
            row_r, _, gb, _, _, sems = bufs
            for j in range(CHUNK_ROWS):
                pltpu.make_async_copy(gb.at[pl.ds(j * LANES, LANES)],
                                      acc.at[row_r.at[j]], sems).wait()

        def min_update(bufs):
            row_r = bufs[0]
            mv = minbuf[...]
            for j in range(CHUNK_ROWS):
                for kk in range(LANES // OUT_C):
                    mv = jnp.minimum(mv, row_r[j, pl.ds(kk * OUT_C, OUT_C)])
            minbuf[...] = mv

        def valid(i):
            return jnp.logical_and(i >= 0, i * 32 + w < FULL_CHUNKS)

        def zfill(i, carry):
            gbuf_a[i, :] = jnp.zeros((OUT_C,), jnp.float32)
            return carry

        lax.fori_loop(0, CHUNK_E, zfill, 0)
        zbase = s * ZROWS
        nfull = ZROWS // CHUNK_E
        for kk in range(nfull):
            pltpu.sync_copy(gbuf_a, acc.at[pl.ds(zbase + kk * CHUNK_E, CHUNK_E)])
        rem = ZROWS % CHUNK_E
        if rem:
            pltpu.sync_copy(gbuf_a.at[pl.ds(0, rem)],
                            acc.at[pl.ds(zbase + nfull * CHUNK_E, rem)])
        plsc.subcore_barrier()

        minbuf[...] = jnp.full((OUT_C,), jnp.int32(N_NODES), jnp.int32)

        fire_idx(0, bufs_a)
        wait_idx(bufs_a)
        fire_gat(bufs_a)

        def pair(g, carry):
            i0 = 2 * g
            i1 = 2 * g + 1

            @pl.when(valid(i1 - 2))
            def _():
                wait_scat(bufs_b)

            @pl.when(valid(i1))
            def _():
                fire_idx(i1, bufs_b)

            @pl.when(valid(i0))
            def _():
                wait_gat(bufs_a)
                min_update(bufs_a)
                fire_scat(bufs_a)

            @pl.when(valid(i1))
            def _():
                wait_idx(bufs_b)
                fire_gat(bufs_b)

            @pl.when(valid(i0))
            def _():
                wait_scat(bufs_a)

            @pl.when(valid(i0 + 2))
            def _():
                fire_idx(i0 + 2, bufs_a)
                wait_idx(bufs_a)
                fire_gat(bufs_a)

            @pl.when(valid(i1))
            def _():
                wait_gat(bufs_b)
                min_update(bufs_b)
                fire_scat(bufs_b)

            return carry

        lax.fori_loop(0, A_PAIRS, pair, 0)

        @pl.when(valid(2 * A_PAIRS - 1))
        def _():
            wait_scat(bufs_b)

        pltpu.sync_copy(minbuf, mins_hbm.at[w])
        plsc.subcore_barrier()

        obase = pl.multiple_of(s * OROWS, 8)

        def copy_out(dst):
            @pl.when(s < 15)
            def _full():
                pltpu.sync_copy(acc.at[pl.ds(obase, OROWS)],
                                dst.at[pl.ds(obase, OROWS)])

            @pl.when(s == 15)
            def _last():
                pltpu.sync_copy(acc.at[pl.ds(15 * OROWS, OROWS_LAST)],
                                dst.at[pl.ds(15 * OROWS, OROWS_LAST)])

        @pl.when(c == 0)
        def _p0():
            copy_out(p0_hbm)

        @pl.when(c == 1)
        def _p1():
            copy_out(p1_hbm)

    return k(edge3, wt)


def _sc_merge(p0, p1, mins, bias):
    mesh = plsc.VectorSubcoreMesh(core_axis_name="c", subcore_axis_name="s")

    @functools.partial(
        pl.kernel,
        mesh=mesh,
        out_type=jax.ShapeDtypeStruct((N_NODES // 8, 8 * OUT_C), jnp.float32),
        scratch_types=[
            pltpu.VMEM((B_CHUNK,), jnp.int32),
            pltpu.VMEM((B_CHUNK,), jnp.int32),
            pltpu.VMEM((B_CHUNK, OUT_C), jnp.float32),
            pltpu.VMEM((B_CHUNK, OUT_C), jnp.float32),
            pltpu.VMEM((B_CHUNK, OUT_C), jnp.float32),
            pltpu.VMEM((B_CHUNK, OUT_C), jnp.float32),
            pltpu.VMEM((B_CHUNK // 8, 8 * OUT_C), jnp.float32),
            pltpu.VMEM((B_CHUNK // 8, 8 * OUT_C), jnp.float32),
            pltpu.VMEM((32, OUT_C), jnp.int32),
            pltpu.VMEM((OUT_C,), jnp.float32),
            pltpu.SemaphoreType.DMA,
            pltpu.SemaphoreType.DMA,
            pltpu.SemaphoreType.DMA,
            pltpu.SemaphoreType.DMA,
        ],
        compiler_params=pltpu.CompilerParams(use_tc_tiling_on_sc=False,
                                             needs_layout_passes=False),
    )
    def k(p0_hbm, p1_hbm, mins_hbm, bias_hbm, out_hbm,
          idx_a, idx_b, b0_a, b1_a, b0_b, b1_b, o_a, o_b, mbuf, bbuf,
          semg_a, semg_b, semo_a, semo_b):
        c = lax.axis_index("c")
        s = lax.axis_index("s")
        w = c * 16 + s

        pltpu.sync_copy(mins_hbm, mbuf)
        pltpu.sync_copy(bias_hbm, bbuf)
        mv = mbuf[0, :]
        for j in range(1, 32):
            mv = jnp.minimum(mv, mbuf[j, :])
        m = jnp.min(mv)
        mvec = jnp.full((OUT_C,), m, jnp.int32)
        bias_v = bbuf[...]
        lane = lax.iota(jnp.int32, OUT_C)

        bufs_a = (idx_a, b0_a, b1_a, o_a, semg_a, semo_a)
        bufs_b = (idx_b, b0_b, b1_b, o_b, semg_b, semo_b)

        def build_idx(i, bufs):
            idx_r = bufs[0]
            base = (i * 32 + w) * B_CHUNK
            for kk in range(B_CHUNK // OUT_C):
                iv = lane + (base + kk * OUT_C) + mvec
                idx_r[pl.ds(kk * OUT_C, OUT_C)] = jnp.minimum(
                    iv, jnp.int32(N_NODES))

        def fire_gat(bufs):
            idx_r, b0, b1, _, semg, _ = bufs
            pltpu.async_copy(p0_hbm.at[idx_r], b0, semg)
            pltpu.async_copy(p1_hbm.at[idx_r], b1, semg)

        def wait_gat(bufs):
            idx_r, b0, b1, _, semg, _ = bufs
            pltpu.make_async_copy(p0_hbm.at[idx_r], b0, semg).wait()
            pltpu.make_async_copy(p1_hbm.at[idx_r], b1, semg).wait()

        def add_rows(bufs):
            _, b0, b1, ob, _, _ = bufs

            def blk(t, carry):
                for r in range(8):
                    kk = t * 8 + r
                    ob[t, pl.ds(r * OUT_C, OUT_C)] = (
                        b0[kk, :] + b1[kk, :] + bias_v)
                return carry

            lax.fori_loop(0, B_CHUNK // 8, blk, 0)

        def fire_out(i, bufs):
            _, _, _, ob, _, semo = bufs
            base = pl.multiple_of((i * 32 + w) * (B_CHUNK // 8), 8)
            pltpu.async_copy(ob, out_hbm.at[pl.ds(base, B_CHUNK // 8)], semo)

        def wait_out(bufs):
            _, _, _, ob, _, semo = bufs
            pltpu.make_async_copy(ob, out_hbm.at[pl.ds(0, B_CHUNK // 8)],
                                  semo).wait()

        def valid(i):
            return jnp.logical_and(i >= 0, i * 32 + w < B_FULL)

        build_idx(0, bufs_a)
        fire_gat(bufs_a)

        def pair(g, carry):
            i0 = 2 * g
            i1 = 2 * g + 1

            @pl.when(valid(i1))
            def _():
                build_idx(i1, bufs_b)

            @pl.when(valid(i1 - 2))
            def _():
                wait_out(bufs_b)

            @pl.when(valid(i1))
            def _():
                fire_gat(bufs_b)

            @pl.when(valid(i0))
            def _():
                wait_gat(bufs_a)
                add_rows(bufs_a)
                fire_out(i0, bufs_a)

            @pl.when(valid(i0 + 2))
            def _():
                build_idx(i0 + 2, bufs_a)

            @pl.when(valid(i0))
            def _():
                wait_out(bufs_a)

            @pl.when(valid(i0 + 2))
            def _():
                fire_gat(bufs_a)

            @pl.when(valid(i1))
            def _():
                wait_gat(bufs_b)
                add_rows(bufs_b)
                fire_out(i1, bufs_b)

            return carry

        lax.fori_loop(0, B_PAIRS, pair, 0)

        @pl.when(valid(2 * B_PAIRS - 1))
        def _():
            wait_out(bufs_b)

        @pl.when(w == 13)
        def _tail():
            base = B_FULL * B_CHUNK
            for kk in range(B_TAIL // OUT_C):
                iv = lane + (base + kk * OUT_C) + mvec
                idx_a[pl.ds(kk * OUT_C, OUT_C)] = jnp.minimum(
                    iv, jnp.int32(N_NODES))
            cp0 = pltpu.async_copy(p0_hbm.at[idx_a.at[pl.ds(0, B_TAIL)]],
                                   b0_a.at[pl.ds(0, B_TAIL)], semg_a)
            cp1 = pltpu.async_copy(p1_hbm.at[idx_a.at[pl.ds(0, B_TAIL)]],
                                   b1_a.at[pl.ds(0, B_TAIL)], semg_a)
            cp0.wait()
            cp1.wait()
            for kk in range(B_TAIL):
                o_a[kk // 8, pl.ds((kk % 8) * OUT_C, OUT_C)] = (
                    b0_a[kk, :] + b1_a[kk, :] + bias_v)
            pltpu.sync_copy(o_a.at[pl.ds(0, B_TAIL // 8)],
                            out_hbm.at[pl.ds(base // 8, B_TAIL // 8)])

    return k(p0, p1, mins, bias)


def kernel(edge_index, W_weight, W_bias):
    edge3 = edge_index.reshape(2, E_ROWS, LANES)
    wt = W_weight.T
    p0, p1, mins = _sc_accumulate(edge3, wt)
    out2d = _sc_merge(p0, p1, mins, W_bias)
    return out2d.reshape(N_NODES, OUT_C)
